# baseline (speedup 1.0000x reference)
import functools

import jax
import jax.numpy as jnp
from jax import lax
from jax.experimental import pallas as pl
from jax.experimental.pallas import tpu as pltpu
from jax.experimental.pallas import tpu_sc as plsc

_NW = 32
_CW = 128
_L = 16


def kernel(embeddings, labels, centers):
    B, D = embeddings.shape
    ch = B // (_NW * _CW)

    emb3d = embeddings.reshape(_NW * ch, _CW, D)
    lab2d = labels.astype(jnp.int32).reshape(_NW * ch, _CW)

    mesh = plsc.VectorSubcoreMesh(core_axis_name="c", subcore_axis_name="s")

    @functools.partial(
        pl.kernel,
        mesh=mesh,
        compiler_params=pltpu.CompilerParams(use_tc_tiling_on_sc=False),
        out_type=jax.ShapeDtypeStruct((_NW, _L), jnp.float32),
        scratch_types=[
            pltpu.VMEM((ch, _CW), jnp.int32),
            pltpu.VMEM((ch, _CW, D), jnp.float32),
            pltpu.VMEM((ch, _CW, D), jnp.float32),
            pltpu.VMEM((_L,), jnp.float32),
            pltpu.SemaphoreType.DMA,
            pltpu.SemaphoreType.DMA,
        ],
    )
    def sc_kernel(emb_hbm, lab_hbm, ctr_hbm, out_hbm,
                  idx_v, emb_v, ctr_v, acc_v, sem_e, sem_g):
        wid = lax.axis_index("s") * 2 + lax.axis_index("c")
        base = wid * ch

        emb_dma = pltpu.async_copy(emb_hbm.at[pl.ds(base, ch)], emb_v, sem_e)
        pltpu.sync_copy(lab_hbm.at[pl.ds(base, ch)], idx_v)
        gathers = [
            pltpu.async_copy(ctr_hbm.at[idx_v.at[j]], ctr_v.at[j], sem_g)
            for j in range(ch)
        ]
        emb_dma.wait()
        for g in gathers:
            g.wait()

        def body(r, acc):
            for j in range(ch):
                for c in range(D // _L):
                    e = emb_v[j, r, pl.ds(c * _L, _L)]
                    t = ctr_v[j, r, pl.ds(c * _L, _L)]
                    d = e - t
                    acc = acc + d * d
            return acc

        acc = lax.fori_loop(0, _CW, body, jnp.zeros((_L,), jnp.float32))
        acc_v[...] = acc
        pltpu.sync_copy(acc_v, out_hbm.at[wid])

    partials = sc_kernel(emb3d, lab2d, centers)
    return partials.sum() / B
